# Initial kernel scaffold; baseline (speedup 1.0000x reference)
import jax
import jax.numpy as jnp
from jax.experimental import pallas as pl


def kernel(x, edge_index, W_emb, b_emb, W1, b1, W2, b2, W3, b3, W_out, b_out):
    raise NotImplementedError("write your pallas kernel here")


# speedup vs baseline: 4.7719x; 4.7719x over previous
import functools

import jax
import jax.numpy as jnp
from jax import lax
from jax.experimental import pallas as pl
from jax.experimental.pallas import tpu as pltpu
from jax.experimental.pallas import tpu_sc as plsc

N = 10000
E = 160000
H = 128
NS = 16
CB = 128
CH = 80
EPAD = NS * CH * CB
RPT = 624
RTAIL = N - NS * RPT
RB = 1000

_f32 = jnp.float32


def _aggr_body(g0, g1, src, dst, out0, out1, src_v, dst_v, rows_v, acc, sem):
    c = lax.axis_index("c")
    s = lax.axis_index("s")
    row0 = s * RPT

    @pl.when(c == 0)
    def _():
        pltpu.sync_copy(g0.at[pl.ds(row0, RPT)], acc.at[pl.ds(row0, RPT)])

    @pl.when(c == 1)
    def _():
        pltpu.sync_copy(g1.at[pl.ds(row0, RPT)], acc.at[pl.ds(row0, RPT)])

    @pl.when(jnp.logical_and(c == 0, s == NS - 1))
    def _():
        pltpu.sync_copy(g0.at[pl.ds(NS * RPT, RTAIL)],
                        acc.at[pl.ds(NS * RPT, RTAIL)])

    @pl.when(jnp.logical_and(c == 1, s == NS - 1))
    def _():
        pltpu.sync_copy(g1.at[pl.ds(NS * RPT, RTAIL)],
                        acc.at[pl.ds(NS * RPT, RTAIL)])

    pltpu.sync_copy(src.at[s], src_v)
    pltpu.sync_copy(dst.at[s], dst_v)
    plsc.subcore_barrier()

    def chunk(j, carry):
        @pl.when(c == 0)
        def _():
            pltpu.async_copy(g0.at[src_v.at[j]], rows_v, sem).wait()

        @pl.when(c == 1)
        def _():
            pltpu.async_copy(g1.at[src_v.at[j]], rows_v, sem).wait()

        pltpu.sync_copy(rows_v, acc.at[dst_v.at[j]], add=True)
        return carry

    lax.fori_loop(0, CH, chunk, 0)
    plsc.subcore_barrier()

    @pl.when(c == 0)
    def _():
        pltpu.sync_copy(acc.at[pl.ds(row0, RPT)], out0.at[pl.ds(row0, RPT)])

    @pl.when(c == 1)
    def _():
        pltpu.sync_copy(acc.at[pl.ds(row0, RPT)], out1.at[pl.ds(row0, RPT)])

    @pl.when(jnp.logical_and(c == 0, s == NS - 1))
    def _():
        pltpu.sync_copy(acc.at[pl.ds(NS * RPT, RTAIL)],
                        out0.at[pl.ds(NS * RPT, RTAIL)])

    @pl.when(jnp.logical_and(c == 1, s == NS - 1))
    def _():
        pltpu.sync_copy(acc.at[pl.ds(NS * RPT, RTAIL)],
                        out1.at[pl.ds(NS * RPT, RTAIL)])


def _aggregate(g0, g1, src, dst):
    mesh = plsc.VectorSubcoreMesh(core_axis_name="c", subcore_axis_name="s")
    f = pl.kernel(
        _aggr_body,
        out_type=(
            jax.ShapeDtypeStruct((N, H), _f32),
            jax.ShapeDtypeStruct((N, H), _f32),
        ),
        mesh=mesh,
        scratch_types=[
            pltpu.VMEM((CH, CB), jnp.int32),
            pltpu.VMEM((CH, CB), jnp.int32),
            pltpu.VMEM((CB, H), _f32),
            pltpu.VMEM_SHARED((N + 8, H), _f32),
            pltpu.SemaphoreType.DMA,
        ],
    )
    return f(g0, g1, src, dst)


def _fold_body(w1_ref, wemb_ref, bemb_ref, m_ref, c_ref):
    m_ref[...] = jnp.dot(w1_ref[...], wemb_ref[...],
                         preferred_element_type=_f32)
    c_ref[...] = lax.dot_general(bemb_ref[...], w1_ref[...],
                                 (((1,), (1,)), ((), ())),
                                 preferred_element_type=_f32)


def _fold(W1, W_emb, b_emb):
    return pl.pallas_call(
        _fold_body,
        out_shape=[
            jax.ShapeDtypeStruct((256, 768), _f32),
            jax.ShapeDtypeStruct((1, 256), _f32),
        ],
    )(W1, W_emb, b_emb.reshape(1, 256))


def _mm_first_body(x_ref, m_ref, c_ref, o0_ref, o1_ref):
    g = lax.dot_general(x_ref[...], m_ref[...], (((1,), (1,)), ((), ())),
                        preferred_element_type=_f32)
    g = g + c_ref[...]
    o0_ref[...] = g[:, :H]
    o1_ref[...] = g[:, H:]


def _mm_first(x, M, c):
    return pl.pallas_call(
        _mm_first_body,
        grid=(N // RB,),
        in_specs=[
            pl.BlockSpec((RB, 768), lambda i: (i, 0)),
            pl.BlockSpec((256, 768), lambda i: (0, 0)),
            pl.BlockSpec((1, 256), lambda i: (0, 0)),
        ],
        out_specs=[pl.BlockSpec((RB, H), lambda i: (i, 0))] * 2,
        out_shape=[jax.ShapeDtypeStruct((N, H), _f32)] * 2,
    )(x, M, c)


def _mm_layer_body(a0_ref, a1_ref, b_ref, w_ref, o0_ref, o1_ref):
    h = jnp.concatenate([a0_ref[...], a1_ref[...]], axis=1) + b_ref[...]
    h = jnp.maximum(h, 0.0)
    g = lax.dot_general(h, w_ref[...], (((1,), (1,)), ((), ())),
                        preferred_element_type=_f32)
    o0_ref[...] = g[:, :H]
    o1_ref[...] = g[:, H:]


def _mm_layer(a0, a1, b, W):
    return pl.pallas_call(
        _mm_layer_body,
        grid=(N // RB,),
        in_specs=[
            pl.BlockSpec((RB, H), lambda i: (i, 0)),
            pl.BlockSpec((RB, H), lambda i: (i, 0)),
            pl.BlockSpec((1, 256), lambda i: (0, 0)),
            pl.BlockSpec((256, 256), lambda i: (0, 0)),
        ],
        out_specs=[pl.BlockSpec((RB, H), lambda i: (i, 0))] * 2,
        out_shape=[jax.ShapeDtypeStruct((N, H), _f32)] * 2,
    )(a0, a1, b.reshape(1, 256), W)


def _final_body(a0_ref, a1_ref, b_ref, wout_ref, bout_ref, o_ref, acc_ref):
    i = pl.program_id(0)
    h = jnp.concatenate([a0_ref[...], a1_ref[...]], axis=1) + b_ref[...]
    h = jnp.maximum(h, 0.0)
    psum = jnp.sum(h, axis=0, keepdims=True)

    @pl.when(i == 0)
    def _():
        acc_ref[...] = psum

    @pl.when(i > 0)
    def _():
        acc_ref[...] = acc_ref[...] + psum

    @pl.when(i == pl.num_programs(0) - 1)
    def _():
        pooled = jnp.maximum(acc_ref[...], 0.0)
        logits = lax.dot_general(pooled, wout_ref[...],
                                 (((1,), (1,)), ((), ())),
                                 preferred_element_type=_f32)
        logits = logits + bout_ref[...]
        m = jnp.max(logits, axis=1, keepdims=True)
        z = logits - m
        o_ref[...] = z - jnp.log(jnp.sum(jnp.exp(z), axis=1, keepdims=True))


def _final(a0, a1, b, W_out, b_out):
    return pl.pallas_call(
        _final_body,
        grid=(N // RB,),
        in_specs=[
            pl.BlockSpec((RB, H), lambda i: (i, 0)),
            pl.BlockSpec((RB, H), lambda i: (i, 0)),
            pl.BlockSpec((1, 256), lambda i: (0, 0)),
            pl.BlockSpec((100, 256), lambda i: (0, 0)),
            pl.BlockSpec((1, 100), lambda i: (0, 0)),
        ],
        out_specs=pl.BlockSpec((1, 100), lambda i: (0, 0)),
        out_shape=jax.ShapeDtypeStruct((1, 100), _f32),
        scratch_shapes=[pltpu.VMEM((1, 256), _f32)],
    )(a0, a1, b.reshape(1, 256), W_out, b_out.reshape(1, 100))


def kernel(x, edge_index, W_emb, b_emb, W1, b1, W2, b2, W3, b3, W_out, b_out):
    npad = EPAD - E
    src = jnp.concatenate(
        [edge_index[0], jnp.zeros((npad,), jnp.int32)]).reshape(NS, CH, CB)
    dst = jnp.concatenate(
        [edge_index[1], jnp.full((npad,), N, jnp.int32)]).reshape(NS, CH, CB)

    M, c1 = _fold(W1, W_emb, b_emb)
    g0, g1 = _mm_first(x, M, c1)
    a0, a1 = _aggregate(g0, g1, src, dst)
    g0, g1 = _mm_layer(a0, a1, b1, W2)
    a0, a1 = _aggregate(g0, g1, src, dst)
    g0, g1 = _mm_layer(a0, a1, b2, W3)
    a0, a1 = _aggregate(g0, g1, src, dst)
    return _final(a0, a1, b3, W_out, b_out)
